# baseline (speedup 1.0000x reference)
import functools

import jax
import jax.numpy as jnp
from jax.experimental import pallas as pl
from jax.experimental.pallas import tpu as pltpu

_S, _B, _C = 7, 2, 20
_D = _B * 5 + _C
_CELLS = _S * _S
_LAMBDA_COORD, _LAMBDA_NOOBJ = 5.0, 0.5
_L2CLAMP = -144.26950408889634


def _reduce_weights():
    r10 = jax.lax.broadcasted_iota(jnp.int32, (8, 10), 0)
    k10 = jax.lax.broadcasted_iota(jnp.int32, (8, 10), 1)
    wsq = ((r10 == 0) & (k10 < 4)) | ((r10 == 1) & (k10 >= 5) & (k10 < 9))
    wsq = wsq | ((r10 == 2) & ((k10 == 4) | (k10 == 9)))
    r20 = jax.lax.broadcasted_iota(jnp.int32, (8, _C), 0)
    return wsq.astype(jnp.bfloat16), (r20 == 0).astype(jnp.bfloat16)


def _loss_body(p_ref, t_ref, o_ref):
    j = pl.program_id(1)
    wsq, wbce = _reduce_weights()
    bnl = o_ref.shape[2]
    zrow = jnp.zeros((1, bnl), jnp.float32)
    acc_coord = zrow
    acc_obj = zrow
    acc_noobj = zrow
    acc_class = zrow

    for c in range(_S):
        t05 = t_ref[c, 0:5]
        tbar = jnp.concatenate([t05, t05], axis=0)
        diff = p_ref[c, 0:10] - tbar
        red = jax.lax.dot_general(
            wsq, (diff * diff).astype(jnp.bfloat16), (((1,), (0,)), ((), ())),
            preferred_element_type=jnp.float32,
        )
        d0 = red[0:1]
        d1 = red[1:2]
        sq01 = red[2:3]

        tw = t_ref[c, 2:3]
        th = t_ref[c, 3:4]
        tconf = t_ref[c, 4:5]
        pw0 = p_ref[c, 2:3]
        ph0 = p_ref[c, 3:4]
        pw1 = p_ref[c, 7:8]
        ph1 = p_ref[c, 8:9]
        i0 = jnp.minimum(pw0, tw) * jnp.minimum(ph0, th)
        i1 = jnp.minimum(pw1, tw) * jnp.minimum(ph1, th)
        tae = tw * th + 1e-6
        u0 = pw0 * ph0 + tae - i0
        u1 = pw1 * ph1 + tae - i1
        swap = i1 * u0 > i0 * u1

        xc = p_ref[c, 10:30]
        yc = t_ref[c, 10:30]
        lg = jnp.maximum(jnp.log2(xc), _L2CLAMP)
        l1 = jnp.maximum(jnp.log2(1.0 - xc), _L2CLAMP)
        bfield = (yc * (lg - l1) + l1).astype(jnp.bfloat16)
        bpos = jax.lax.dot_general(
            wbce, bfield, (((1,), (0,)), ((), ())),
            preferred_element_type=jnp.float32,
        )[0:1]

        acc_coord += tconf * jnp.where(swap, d1, d0)
        objrow = tconf * sq01
        acc_obj += objrow
        acc_noobj += sq01 - objrow
        acc_class += tconf * bpos

    @pl.when(j == 0)
    def _first():
        o_ref[0, 0:1, :] = acc_coord
        o_ref[0, 1:2, :] = acc_obj
        o_ref[0, 2:3, :] = acc_noobj
        o_ref[0, 3:4, :] = acc_class

    @pl.when(j != 0)
    def _rest():
        o_ref[0, 0:1, :] += acc_coord
        o_ref[0, 1:2, :] += acc_obj
        o_ref[0, 2:3, :] += acc_noobj
        o_ref[0, 3:4, :] += acc_class


@functools.partial(jax.jit, static_argnames=("bnl",))
def _yolo_loss(predictions, targets, bnl=2048):
    n = predictions.shape[0]
    p = jnp.transpose(predictions, (1, 2, 3, 0)).reshape(_CELLS, _D, n)
    t = jnp.transpose(targets, (1, 2, 3, 0)).reshape(_CELLS, _D, n)
    nb = n // bnl
    partial = pl.pallas_call(
        _loss_body,
        grid=(nb, _S),
        in_specs=[
            pl.BlockSpec((_S, _D, bnl), lambda i, j: (j, 0, i)),
            pl.BlockSpec((_S, _D, bnl), lambda i, j: (j, 0, i)),
        ],
        out_specs=pl.BlockSpec((1, 4, bnl), lambda i, j: (i, 0, 0)),
        out_shape=jax.ShapeDtypeStruct((nb, 4, bnl), jnp.float32),
        compiler_params=pltpu.CompilerParams(
            dimension_semantics=("arbitrary", "arbitrary"),
        ),
    )(p, t)
    sums = jnp.sum(partial, axis=(0, 2))
    ln2 = 0.6931471805599453
    coord, objl, nobjl, classl = sums[0], sums[1], sums[2], -ln2 * sums[3]
    total = (_LAMBDA_COORD * coord + objl + _LAMBDA_NOOBJ * nobjl + classl) / n
    return (total, coord / n, objl / n, nobjl / n, classl / n)


def kernel(predictions, targets):
    return _yolo_loss(predictions, targets)


# speedup vs baseline: 1.2654x; 1.2654x over previous
import functools

import jax
import jax.numpy as jnp
from jax.experimental import pallas as pl
from jax.experimental.pallas import tpu as pltpu

_S, _B, _C = 7, 2, 20
_D = _B * 5 + _C
_CELLS = _S * _S
_LAMBDA_COORD, _LAMBDA_NOOBJ = 5.0, 0.5
_L2CLAMP = -144.26950408889634


def _reduce_weights():
    r10 = jax.lax.broadcasted_iota(jnp.int32, (8, 10), 0)
    k10 = jax.lax.broadcasted_iota(jnp.int32, (8, 10), 1)
    wsq = ((r10 == 0) & (k10 < 4)) | ((r10 == 1) & (k10 >= 5) & (k10 < 9))
    wsq = wsq | ((r10 == 2) & ((k10 == 4) | (k10 == 9)))
    r20 = jax.lax.broadcasted_iota(jnp.int32, (8, _C), 0)
    return wsq.astype(jnp.bfloat16), (r20 == 0).astype(jnp.bfloat16)


def _loss_body(p_ref, t_ref, o_ref):
    wsq, wbce = _reduce_weights()
    bnl = o_ref.shape[2]
    zrow = jnp.zeros((1, bnl), jnp.float32)
    acc_coord = zrow
    acc_obj = zrow
    acc_noobj = zrow
    acc_class = zrow

    for c in range(_CELLS):
        t05 = t_ref[c, 0:5]
        tbar = jnp.concatenate([t05, t05], axis=0)
        diff = p_ref[c, 0:10] - tbar
        red = jax.lax.dot_general(
            wsq, (diff * diff).astype(jnp.bfloat16), (((1,), (0,)), ((), ())),
            preferred_element_type=jnp.float32,
        )
        d0 = red[0:1]
        d1 = red[1:2]
        sq01 = red[2:3]

        tw = t_ref[c, 2:3]
        th = t_ref[c, 3:4]
        tconf = t_ref[c, 4:5]
        pw0 = p_ref[c, 2:3]
        ph0 = p_ref[c, 3:4]
        pw1 = p_ref[c, 7:8]
        ph1 = p_ref[c, 8:9]
        i0 = jnp.minimum(pw0, tw) * jnp.minimum(ph0, th)
        i1 = jnp.minimum(pw1, tw) * jnp.minimum(ph1, th)
        tae = tw * th + 1e-6
        u0 = pw0 * ph0 + tae - i0
        u1 = pw1 * ph1 + tae - i1
        swap = i1 * u0 > i0 * u1

        xc = p_ref[c, 10:30]
        yc = t_ref[c, 10:30]
        lg = jnp.maximum(jnp.log2(xc), _L2CLAMP)
        l1 = jnp.maximum(jnp.log2(1.0 - xc), _L2CLAMP)
        bfield = (yc * (lg - l1) + l1).astype(jnp.bfloat16)
        bpos = jax.lax.dot_general(
            wbce, bfield, (((1,), (0,)), ((), ())),
            preferred_element_type=jnp.float32,
        )[0:1]

        acc_coord += tconf * jnp.where(swap, d1, d0)
        objrow = tconf * sq01
        acc_obj += objrow
        acc_noobj += sq01 - objrow
        acc_class += tconf * bpos

    o_ref[0, 0:1, :] = acc_coord
    o_ref[0, 1:2, :] = acc_obj
    o_ref[0, 2:3, :] = acc_noobj
    o_ref[0, 3:4, :] = acc_class


@functools.partial(jax.jit, static_argnames=("bnl",))
def _yolo_loss(predictions, targets, bnl=2048):
    n = predictions.shape[0]
    p = jnp.transpose(predictions, (1, 2, 3, 0)).reshape(_CELLS, _D, n)
    t = jnp.transpose(targets, (1, 2, 3, 0)).reshape(_CELLS, _D, n)
    nb = n // bnl
    partial = pl.pallas_call(
        _loss_body,
        grid=(nb,),
        in_specs=[
            pl.BlockSpec((_CELLS, _D, bnl), lambda i: (0, 0, i)),
            pl.BlockSpec((_CELLS, _D, bnl), lambda i: (0, 0, i)),
        ],
        out_specs=pl.BlockSpec((1, 4, bnl), lambda i: (i, 0, 0)),
        out_shape=jax.ShapeDtypeStruct((nb, 4, bnl), jnp.float32),
        compiler_params=pltpu.CompilerParams(
            dimension_semantics=("arbitrary",),
        ),
    )(p, t)
    sums = jnp.sum(partial, axis=(0, 2))
    ln2 = 0.6931471805599453
    coord, objl, nobjl, classl = sums[0], sums[1], sums[2], -ln2 * sums[3]
    total = (_LAMBDA_COORD * coord + objl + _LAMBDA_NOOBJ * nobjl + classl) / n
    return (total, coord / n, objl / n, nobjl / n, classl / n)


def kernel(predictions, targets):
    return _yolo_loss(predictions, targets)


# speedup vs baseline: 1.4721x; 1.1634x over previous
import functools

import jax
import jax.numpy as jnp
from jax.experimental import pallas as pl
from jax.experimental.pallas import tpu as pltpu

_S, _B, _C = 7, 2, 20
_D = _B * 5 + _C
_CELLS = _S * _S
_LAMBDA_COORD, _LAMBDA_NOOBJ = 5.0, 0.5
_L2CLAMP = -144.26950408889634


def _reduce_weights():
    r10 = jax.lax.broadcasted_iota(jnp.int32, (8, 10), 0)
    k10 = jax.lax.broadcasted_iota(jnp.int32, (8, 10), 1)
    wsq = ((r10 == 0) & (k10 < 4)) | ((r10 == 1) & (k10 >= 5) & (k10 < 9))
    wsq = wsq | ((r10 == 2) & ((k10 == 4) | (k10 == 9)))
    r20 = jax.lax.broadcasted_iota(jnp.int32, (8, _C), 0)
    return wsq.astype(jnp.bfloat16), (r20 == 0).astype(jnp.bfloat16)


def _loss_body(p_ref, t_ref, o_total, o_coord, o_obj, o_noobj, o_class):
    i = pl.program_id(0)
    nb = pl.num_programs(0)
    wsq, wbce = _reduce_weights()
    bnl = p_ref.shape[2]
    zrow = jnp.zeros((1, bnl), jnp.float32)
    acc_coord = zrow
    acc_obj = zrow
    acc_noobj = zrow
    acc_class = zrow

    for c in range(_CELLS):
        t05 = t_ref[c, 0:5]
        tbar = jnp.concatenate([t05, t05], axis=0)
        diff = p_ref[c, 0:10] - tbar
        red = jax.lax.dot_general(
            wsq, (diff * diff).astype(jnp.bfloat16), (((1,), (0,)), ((), ())),
            preferred_element_type=jnp.float32,
        )
        d0 = red[0:1]
        d1 = red[1:2]
        sq01 = red[2:3]

        tw = t_ref[c, 2:3]
        th = t_ref[c, 3:4]
        tconf = t_ref[c, 4:5]
        pw0 = p_ref[c, 2:3]
        ph0 = p_ref[c, 3:4]
        pw1 = p_ref[c, 7:8]
        ph1 = p_ref[c, 8:9]
        i0 = jnp.minimum(pw0, tw) * jnp.minimum(ph0, th)
        i1 = jnp.minimum(pw1, tw) * jnp.minimum(ph1, th)
        tae = tw * th + 1e-6
        u0 = pw0 * ph0 + tae - i0
        u1 = pw1 * ph1 + tae - i1
        swap = i1 * u0 > i0 * u1

        xc = p_ref[c, 10:30]
        yc = t_ref[c, 10:30]
        lg = jnp.maximum(jnp.log2(xc), _L2CLAMP)
        l1 = jnp.maximum(jnp.log2(1.0 - xc), _L2CLAMP)
        bfield = (yc * (lg - l1) + l1).astype(jnp.bfloat16)
        bpos = jax.lax.dot_general(
            wbce, bfield, (((1,), (0,)), ((), ())),
            preferred_element_type=jnp.float32,
        )[0:1]

        acc_coord += tconf * jnp.where(swap, d1, d0)
        objrow = tconf * sq01
        acc_obj += objrow
        acc_noobj += sq01 - objrow
        acc_class += tconf * bpos

    n = nb * bnl
    cs = jnp.sum(acc_coord) / n
    os_ = jnp.sum(acc_obj) / n
    ns = jnp.sum(acc_noobj) / n
    ks = -0.6931471805599453 * jnp.sum(acc_class) / n

    @pl.when(i == 0)
    def _first():
        o_coord[0] = cs
        o_obj[0] = os_
        o_noobj[0] = ns
        o_class[0] = ks

    @pl.when(i != 0)
    def _rest():
        o_coord[0] += cs
        o_obj[0] += os_
        o_noobj[0] += ns
        o_class[0] += ks

    @pl.when(i == nb - 1)
    def _final():
        o_total[0] = (_LAMBDA_COORD * o_coord[0] + o_obj[0]
                      + _LAMBDA_NOOBJ * o_noobj[0] + o_class[0])


@functools.partial(jax.jit, static_argnames=("bnl",))
def _yolo_loss(predictions, targets, bnl=2048):
    n = predictions.shape[0]
    p = jnp.transpose(predictions, (1, 2, 3, 0)).reshape(_CELLS, _D, n)
    t = jnp.transpose(targets, (1, 2, 3, 0)).reshape(_CELLS, _D, n)
    nb = n // bnl
    scalar = jax.ShapeDtypeStruct((1,), jnp.float32)
    smem = pl.BlockSpec(memory_space=pltpu.SMEM)
    total, coord, objl, nobjl, classl = pl.pallas_call(
        _loss_body,
        grid=(nb,),
        in_specs=[
            pl.BlockSpec((_CELLS, _D, bnl), lambda i: (0, 0, i)),
            pl.BlockSpec((_CELLS, _D, bnl), lambda i: (0, 0, i)),
        ],
        out_specs=[smem, smem, smem, smem, smem],
        out_shape=[scalar, scalar, scalar, scalar, scalar],
        compiler_params=pltpu.CompilerParams(
            dimension_semantics=("arbitrary",),
        ),
    )(p, t)
    return (total.reshape(()), coord.reshape(()), objl.reshape(()),
            nobjl.reshape(()), classl.reshape(()))


def kernel(predictions, targets):
    return _yolo_loss(predictions, targets)
